# Initial kernel scaffold; baseline (speedup 1.0000x reference)
import jax
import jax.numpy as jnp
from jax.experimental import pallas as pl


def kernel(node_features, row_ppi, col_ppi, score_ppi, row_path, col_path, score_path, row_go, col_go, score_go, enc_W1, enc_b1, enc_W2, enc_b2, enc_W3, enc_b3, cls_W1, cls_b1, cls_W2, cls_b2, proj_W1, proj_b1, proj_W2, proj_b2, att_W1, att_b1, att_W2, att_b2, fus_W1, fus_b1, fus_W2, fus_b2):
    raise NotImplementedError("write your pallas kernel here")


# speedup vs baseline: 6.0129x; 6.0129x over previous
import functools

import jax
import jax.numpy as jnp
from jax import lax
from jax.experimental import pallas as pl
from jax.experimental.pallas import tpu as pltpu
from jax.experimental.pallas import tpu_sc as plsc

N = 10000
NP = 10112
V = 3
NC = 2
NS = 16
NW = NC * NS
EB = 128

_HI = lax.Precision.DEFAULT


@functools.cache
def _make_spmm(n_rows, k, e_seg, n_seg=1):
    per_tile = e_seg // NW
    n_batches = per_tile // EB
    rpt = n_rows // NS
    mesh = plsc.VectorSubcoreMesh(core_axis_name="c", subcore_axis_name="s")

    @functools.partial(
        pl.kernel,
        out_type=jax.ShapeDtypeStruct((n_seg, NC, n_rows, k), jnp.float32),
        mesh=mesh,
        scratch_types=[
            pltpu.VMEM((EB,), jnp.int32),
            pltpu.VMEM((EB,), jnp.int32),
            pltpu.VMEM((EB,), jnp.float32),
            pltpu.VMEM((EB, k), jnp.float32),
            pltpu.VMEM_SHARED((n_rows, k), jnp.float32),
            pltpu.SemaphoreType.DMA,
        ],
        compiler_params=pltpu.CompilerParams(use_tc_tiling_on_sc=False),
    )
    def spmm(x_hbm, col_hbm, row_hbm, score_hbm, zeros_hbm, out_hbm,
             col_v, row_v, score_v, rows_v, acc_sh, sem):
        c = lax.axis_index("c")
        s = lax.axis_index("s")
        wid = c * NS + s

        for seg in range(n_seg):
            pltpu.sync_copy(zeros_hbm, acc_sh.at[pl.ds(s * rpt, rpt)])
            plsc.subcore_barrier()

            base0 = seg * e_seg + wid * per_tile

            def body(b, carry):
                base = base0 + b * EB
                pltpu.sync_copy(col_hbm.at[pl.ds(base, EB)], col_v)
                pltpu.sync_copy(row_hbm.at[pl.ds(base, EB)], row_v)
                pltpu.sync_copy(score_hbm.at[pl.ds(base, EB)], score_v)
                pltpu.async_copy(x_hbm.at[col_v], rows_v, sem).wait()

                def scale(g, c2):
                    svec = score_v[pl.ds(g * 16, 16)]
                    for e in range(16):
                        lane = jnp.full((16,), e, jnp.int32)
                        sv = svec.at[lane].get(mode="promise_in_bounds")
                        row = g * 16 + e
                        for j in range(k // 16):
                            sl = pl.ds(j * 16, 16)
                            rows_v[row, sl] = rows_v[row, sl] * sv
                    return c2

                lax.fori_loop(0, EB // 16, scale, 0)
                pltpu.sync_copy(rows_v, acc_sh.at[row_v], add=True)
                return carry

            lax.fori_loop(0, n_batches, body, 0)
            plsc.subcore_barrier()
            pltpu.sync_copy(acc_sh.at[pl.ds(s * rpt, rpt)],
                            out_hbm.at[seg, c, pl.ds(s * rpt, rpt), :])

    return spmm


def _pad_edges(col, row, score, mult):
    e = col.shape[0]
    pad = (-e) % mult
    if pad:
        z = jnp.zeros((pad,), jnp.int32)
        col = jnp.concatenate([col, z])
        row = jnp.concatenate([row, z])
        score = jnp.concatenate([score, jnp.zeros((pad,), score.dtype)])
    return col, row, score


RB = 1000


def _full(spec_shape):
    r = len(spec_shape)
    return pl.BlockSpec(spec_shape, lambda i: (0,) * r)


def _rsqrt(x):
    r = lax.rsqrt(x)
    r = r * (1.5 - 0.5 * x * r * r)
    r = r * (1.5 - 0.5 * x * r * r)
    return r


def _prep_body(degp_ref, nf_ref, w1_ref, b1_ref, dis_ref, xs1_ref):
    deg = degp_ref[0, :, :, 0] + degp_ref[1, :, :, 0] + 1.0
    dis = _rsqrt(jnp.maximum(deg, 1e-12))
    dis_ref[...] = dis[:, :, None]
    nf = nf_ref[...]
    for v in range(V):
        x1 = jnp.dot(nf, w1_ref[v].T, precision=_HI) + b1_ref[v]
        xs1_ref[v] = dis[v][:, None] * x1


def _prep(degp, nf, w1, b1):
    g = N // RB
    return pl.pallas_call(
        _prep_body,
        grid=(g,),
        in_specs=[
            pl.BlockSpec((NC, V, RB, 16), lambda i: (0, 0, i, 0)),
            pl.BlockSpec((RB, 128), lambda i: (i, 0)),
            _full(w1.shape),
            _full(b1.shape),
        ],
        out_specs=[
            pl.BlockSpec((V, RB, 1), lambda i: (0, i, 0)),
            pl.BlockSpec((V, RB, 128), lambda i: (0, i, 0)),
        ],
        out_shape=[
            jax.ShapeDtypeStruct((V, N, 1), jnp.float32),
            jax.ShapeDtypeStruct((V, N, 128), jnp.float32),
        ],
    )(degp, nf, w1, b1)


def _combine_body(p_ref, xs_ref, dis_ref, w_ref, b_ref, out_ref):
    for v in range(V):
        sacc = p_ref[0, v] + p_ref[1, v] + xs_ref[v]
        h = dis_ref[v] * sacc
        h = jnp.where(h > 0, h, 0.2 * h)
        out_ref[v] = dis_ref[v] * (
            jnp.dot(h, w_ref[v].T, precision=_HI) + b_ref[v])


def _combine(partials, xs, dis, w, b):
    g = N // RB
    k = xs.shape[-1]
    k2 = w.shape[1]
    return pl.pallas_call(
        _combine_body,
        grid=(g,),
        in_specs=[
            pl.BlockSpec((NC, V, RB, k), lambda i: (0, 0, i, 0)),
            pl.BlockSpec((V, RB, k), lambda i: (0, i, 0)),
            pl.BlockSpec((V, RB, 1), lambda i: (0, i, 0)),
            _full(w.shape),
            _full(b.shape),
        ],
        out_specs=pl.BlockSpec((V, RB, k2), lambda i: (0, i, 0)),
        out_shape=jax.ShapeDtypeStruct((V, N, k2), jnp.float32),
    )(partials, xs, dis, w, b)


def _pad_w1(w_row):
    ch = w_row.shape[-1]
    return jnp.concatenate(
        [jnp.reshape(w_row, (ch, 1)), jnp.zeros((ch, 127), jnp.float32)], axis=1)


def _finish_body(p_ref, xs_ref, dis_ref,
                 cw1_ref, cb1_ref, cw2_ref, cb2_ref,
                 pw1_ref, pb1_ref, pw2_ref, pb2_ref,
                 aw1_ref, ab1_ref, aw2_ref, ab2_ref,
                 fw1_ref, fb1_ref, fw2_ref, fb2_ref,
                 z_ref, pz_ref, logit_ref, fused_ref, flogit_ref, att_ref):
    zs = []
    for v in range(V):
        sacc = p_ref[0, v] + p_ref[1, v] + xs_ref[v]
        z = dis_ref[v] * sacc
        zs.append(z)
        z_ref[v] = z
        hc = jax.nn.relu(jnp.dot(z, cw1_ref[v].T, precision=_HI) + cb1_ref[v])
        logit_ref[v] = jnp.dot(hc, _pad_w1(cw2_ref[v]), precision=_HI)[:, 0:1] + cb2_ref[v, 0]
        hp = jax.nn.relu(jnp.dot(z, pw1_ref[v].T, precision=_HI) + pb1_ref[v])
        pz = jnp.dot(hp, pw2_ref[v].T, precision=_HI) + pb2_ref[v]
        ss = jnp.maximum(jnp.sum(pz * pz, axis=-1, keepdims=True), 1e-24)
        pz_ref[v] = pz * _rsqrt(ss)
    concat = jnp.concatenate(zs, axis=-1)
    ha = jax.nn.relu(jnp.dot(concat, aw1_ref[...].T, precision=_HI) + ab1_ref[...])
    alog = jnp.dot(ha, aw2_ref[...].T, precision=_HI) + ab2_ref[...]
    am = jnp.max(alog, axis=-1, keepdims=True)
    ae = jnp.exp(alog - am)
    att = ae / jnp.sum(ae, axis=-1, keepdims=True)
    att_ref[...] = att
    fused = (zs[0] * att[:, 0:1] + zs[1] * att[:, 1:2] + zs[2] * att[:, 2:3])
    fused_ref[...] = fused
    hf = jax.nn.relu(jnp.dot(fused, fw1_ref[...].T, precision=_HI) + fb1_ref[...])
    flogit_ref[...] = jnp.dot(hf, _pad_w1(fw2_ref[...]), precision=_HI)[:, 0:1] + fb2_ref[0]


def _finish(partials, xs, dis, cw1, cb1, cw2, cb2, pw1, pb1, pw2, pb2,
            aw1, ab1, aw2, ab2, fw1, fb1, fw2, fb2):
    g = N // RB
    k = xs.shape[-1]
    ws = [cw1, cb1, cw2, cb2, pw1, pb1, pw2, pb2, aw1, ab1, aw2, ab2,
          fw1, fb1, fw2, fb2]
    return pl.pallas_call(
        _finish_body,
        grid=(g,),
        in_specs=[
            pl.BlockSpec((NC, V, RB, k), lambda i: (0, 0, i, 0)),
            pl.BlockSpec((V, RB, k), lambda i: (0, i, 0)),
            pl.BlockSpec((V, RB, 1), lambda i: (0, i, 0)),
        ] + [_full(w.shape) for w in ws],
        out_specs=[
            pl.BlockSpec((V, RB, k), lambda i: (0, i, 0)),
            pl.BlockSpec((V, RB, k), lambda i: (0, i, 0)),
            pl.BlockSpec((V, RB, 1), lambda i: (0, i, 0)),
            pl.BlockSpec((RB, k), lambda i: (i, 0)),
            pl.BlockSpec((RB, 1), lambda i: (i, 0)),
            pl.BlockSpec((RB, V), lambda i: (i, 0)),
        ],
        out_shape=[
            jax.ShapeDtypeStruct((V, N, k), jnp.float32),
            jax.ShapeDtypeStruct((V, N, k), jnp.float32),
            jax.ShapeDtypeStruct((V, N, 1), jnp.float32),
            jax.ShapeDtypeStruct((N, k), jnp.float32),
            jax.ShapeDtypeStruct((N, 1), jnp.float32),
            jax.ShapeDtypeStruct((N, V), jnp.float32),
        ],
    )(partials, xs, dis, *ws)


def kernel(node_features, row_ppi, col_ppi, score_ppi, row_path, col_path,
           score_path, row_go, col_go, score_go, enc_W1, enc_b1, enc_W2,
           enc_b2, enc_W3, enc_b3, cls_W1, cls_b1, cls_W2, cls_b2, proj_W1,
           proj_b1, proj_W2, proj_b2, att_W1, att_b1, att_W2, att_b2,
           fus_W1, fus_b1, fus_W2, fus_b2):
    rows = [row_ppi, row_path, row_go]
    cols = [col_ppi, col_path, col_go]
    scores = [score_ppi, score_path, score_go]
    mult = NW * EB

    ev = [_pad_edges(cols[v] + v * N, rows[v], scores[v], mult)
          for v in range(V)]
    col1 = jnp.concatenate([e[0] for e in ev])
    row1 = jnp.concatenate([e[1] for e in ev])
    score1 = jnp.concatenate([e[2] for e in ev])
    col_b, row_b, score_b = _pad_edges(
        jnp.concatenate([cols[v] + v * NP for v in range(V)]),
        jnp.concatenate([rows[v] + v * NP for v in range(V)]),
        jnp.concatenate(scores), mult)
    e1 = ev[0][0].shape[0]
    eb = col_b.shape[0]

    zeros16 = jnp.zeros((V * NP // NS, 16), jnp.float32)
    zeros64 = jnp.zeros((V * NP // NS, 64), jnp.float32)
    zeros128 = jnp.zeros((NP // NS, 128), jnp.float32)
    ones = jnp.ones((V * NP, 16), jnp.float32)

    def _padr(x):
        return jnp.pad(x, ((0, 0), (0, NP - N), (0, 0))).reshape(V * NP, -1)

    degp = _make_spmm(V * NP, 16, eb)(ones, col_b, row_b, score_b, zeros16)
    degp = degp[0].reshape(NC, V, NP, 16)[:, :, :N]

    dis, xs1 = _prep(degp, node_features, enc_W1, enc_b1)

    p1 = _make_spmm(NP, 128, e1, V)(
        xs1.reshape(V * N, 128), col1, row1, score1, zeros128)
    p1 = jnp.moveaxis(p1[:, :, :N], 0, 1)

    xs2 = _combine(p1, xs1, dis, enc_W2, enc_b2)

    p2 = _make_spmm(V * NP, 64, eb)(
        _padr(xs2), col_b, row_b, score_b, zeros64)
    xs3 = _combine(p2[0].reshape(NC, V, NP, 64)[:, :, :N], xs2, dis,
                   enc_W3, enc_b3)

    p3 = _make_spmm(V * NP, 64, eb)(
        _padr(xs3), col_b, row_b, score_b, zeros64)

    z, pz, logits, fused, flogit, att = _finish(
        p3[0].reshape(NC, V, NP, 64)[:, :, :N], xs3, dis,
        cls_W1, cls_b1, cls_W2, cls_b2, proj_W1, proj_b1, proj_W2, proj_b2,
        att_W1, att_b1, att_W2, att_b2, fus_W1, fus_b1, fus_W2, fus_b2)

    return (z[0], z[1], z[2], pz[0], pz[1], pz[2],
            logits[0, :, 0], logits[1, :, 0], logits[2, :, 0],
            fused, flogit[:, 0], att)


# speedup vs baseline: 6.6118x; 1.0996x over previous
import functools

import jax
import jax.numpy as jnp
from jax import lax
from jax.experimental import pallas as pl
from jax.experimental.pallas import tpu as pltpu
from jax.experimental.pallas import tpu_sc as plsc

N = 10000
NP = 10112
V = 3
NC = 2
NS = 16
NW = NC * NS
EB = 128

_HI = lax.Precision.DEFAULT


def _group_size(k):
    return {128: 2, 64: 4}.get(k, 8)


@functools.cache
def _make_spmm(n_rows, k, e_seg, n_seg=1):
    G = _group_size(k)
    per_tile = e_seg // NW
    nbt = per_tile // EB
    ng = nbt // G
    assert nbt % G == 0 and ng % 2 == 0 and ng >= 4
    rpt = n_rows // NS
    mesh = plsc.VectorSubcoreMesh(core_axis_name="c", subcore_axis_name="s")

    @functools.partial(
        pl.kernel,
        out_type=jax.ShapeDtypeStruct((n_seg, NC, n_rows, k), jnp.float32),
        mesh=mesh,
        scratch_types=[
            pltpu.VMEM((3 * G, EB), jnp.int32),
            pltpu.VMEM((3 * G, EB), jnp.int32),
        ] + [pltpu.VMEM((EB, k), jnp.float32) for _ in range(G)] + [
            pltpu.SemaphoreType.DMA((G,)),
            pltpu.SemaphoreType.DMA((G,)),
            pltpu.SemaphoreType.DMA((2,)),
            pltpu.VMEM_SHARED((n_rows, k), jnp.float32),
        ],
        compiler_params=pltpu.CompilerParams(use_tc_tiling_on_sc=False),
    )
    def spmm(x_hbm, edges_hbm, zeros_hbm, out_hbm, eb0, eb1, *rest):
        rows = rest[:G]
        gsem, ssem, esem, acc_sh = rest[G:]
        c = lax.axis_index("c")
        s = lax.axis_index("s")
        wid = c * NS + s

        def scale(buf, j):
            def body16(g16, c2):
                svec = lax.bitcast_convert_type(
                    buf[3 * j + 2, pl.ds(g16 * 16, 16)], jnp.float32)
                for e in range(16):
                    lane = jnp.full((16,), e, jnp.int32)
                    sv = svec.at[lane].get(mode="promise_in_bounds")
                    r = g16 * 16 + e
                    for jj in range(k // 16):
                        sl = pl.ds(jj * 16, 16)
                        rows[j][r, sl] = rows[j][r, sl] * sv
                return c2
            lax.fori_loop(0, EB // 16, body16, 0)

        def do_group(buf, first):
            descs = []
            for j in range(G):
                if not first:
                    pltpu.make_async_copy(
                        rows[j], acc_sh.at[buf.at[3 * j + 1]],
                        ssem.at[j]).wait()
                descs.append(pltpu.async_copy(
                    x_hbm.at[buf.at[3 * j]], rows[j], gsem.at[j]))
            for j in range(G):
                descs[j].wait()
                scale(buf, j)
                pltpu.async_copy(rows[j], acc_sh.at[buf.at[3 * j + 1]],
                                 ssem.at[j], add=True)

        for seg in range(n_seg):
            pltpu.sync_copy(zeros_hbm, acc_sh.at[pl.ds(s * rpt, rpt)])
            plsc.subcore_barrier()

            b0 = (seg * NW + wid) * nbt

            def edge_issue(grp, buf, p):
                return pltpu.async_copy(
                    edges_hbm.at[pl.ds((b0 + grp * G) * 3, 3 * G)],
                    buf, esem.at[p])

            def edge_wait(buf, p):
                pltpu.make_async_copy(
                    edges_hbm.at[pl.ds(b0 * 3, 3 * G)], buf,
                    esem.at[p]).wait()

            pltpu.sync_copy(edges_hbm.at[pl.ds(b0 * 3, 3 * G)], eb0)
            edge_issue(1, eb1, 1)
            do_group(eb0, first=True)
            edge_issue(2, eb0, 0)

            def pair(i, carry):
                g1 = 1 + 2 * i
                edge_wait(eb1, 1)
                do_group(eb1, first=False)

                @pl.when(g1 + 2 < ng)
                def _():
                    edge_issue(g1 + 2, eb1, 1)

                edge_wait(eb0, 0)
                do_group(eb0, first=False)

                @pl.when(g1 + 3 < ng)
                def _():
                    edge_issue(g1 + 3, eb0, 0)
                return carry

            lax.fori_loop(0, (ng - 2) // 2, pair, 0)

            edge_wait(eb1, 1)
            do_group(eb1, first=False)
            for j in range(G):
                pltpu.make_async_copy(
                    rows[j], acc_sh.at[eb1.at[3 * j + 1]], ssem.at[j]).wait()

            plsc.subcore_barrier()
            pltpu.sync_copy(acc_sh.at[pl.ds(s * rpt, rpt)],
                            out_hbm.at[seg, c, pl.ds(s * rpt, rpt), :])

    return spmm


def _pad_edges(col, row, score, mult):
    e = col.shape[0]
    pad = (-e) % mult
    if pad:
        z = jnp.zeros((pad,), jnp.int32)
        col = jnp.concatenate([col, z])
        row = jnp.concatenate([row, z])
        score = jnp.concatenate([score, jnp.zeros((pad,), score.dtype)])
    return col, row, score


def _pack_edges(col, row, score):
    nb = col.shape[0] // EB
    packed = jnp.stack([
        col.reshape(nb, EB), row.reshape(nb, EB),
        lax.bitcast_convert_type(score, jnp.int32).reshape(nb, EB)], axis=1)
    return packed.reshape(nb * 3, EB)


RB = 1000


def _full(spec_shape):
    r = len(spec_shape)
    return pl.BlockSpec(spec_shape, lambda i: (0,) * r)


def _rsqrt(x):
    r = lax.rsqrt(x)
    r = r * (1.5 - 0.5 * x * r * r)
    r = r * (1.5 - 0.5 * x * r * r)
    return r


def _prep_body(degp_ref, nf_ref, w1_ref, b1_ref, dis_ref, xs1_ref):
    deg = degp_ref[0, :, :, 0] + degp_ref[1, :, :, 0] + 1.0
    dis = _rsqrt(jnp.maximum(deg, 1e-12))
    dis_ref[...] = dis[:, :, None]
    nf = nf_ref[...]
    for v in range(V):
        x1 = jnp.dot(nf, w1_ref[v].T, precision=_HI) + b1_ref[v]
        xs1_ref[v] = dis[v][:, None] * x1


def _prep(degp, nf, w1, b1):
    g = N // RB
    return pl.pallas_call(
        _prep_body,
        grid=(g,),
        in_specs=[
            pl.BlockSpec((NC, V, RB, 16), lambda i: (0, 0, i, 0)),
            pl.BlockSpec((RB, 128), lambda i: (i, 0)),
            _full(w1.shape),
            _full(b1.shape),
        ],
        out_specs=[
            pl.BlockSpec((V, RB, 1), lambda i: (0, i, 0)),
            pl.BlockSpec((V, RB, 128), lambda i: (0, i, 0)),
        ],
        out_shape=[
            jax.ShapeDtypeStruct((V, N, 1), jnp.float32),
            jax.ShapeDtypeStruct((V, N, 128), jnp.float32),
        ],
    )(degp, nf, w1, b1)


def _combine_body(p_ref, xs_ref, dis_ref, w_ref, b_ref, out_ref):
    for v in range(V):
        sacc = p_ref[0, v] + p_ref[1, v] + xs_ref[v]
        h = dis_ref[v] * sacc
        h = jnp.where(h > 0, h, 0.2 * h)
        out_ref[v] = dis_ref[v] * (
            jnp.dot(h, w_ref[v].T, precision=_HI) + b_ref[v])


def _combine(partials, xs, dis, w, b):
    g = N // RB
    k = xs.shape[-1]
    k2 = w.shape[1]
    return pl.pallas_call(
        _combine_body,
        grid=(g,),
        in_specs=[
            pl.BlockSpec((NC, V, RB, k), lambda i: (0, 0, i, 0)),
            pl.BlockSpec((V, RB, k), lambda i: (0, i, 0)),
            pl.BlockSpec((V, RB, 1), lambda i: (0, i, 0)),
            _full(w.shape),
            _full(b.shape),
        ],
        out_specs=pl.BlockSpec((V, RB, k2), lambda i: (0, i, 0)),
        out_shape=jax.ShapeDtypeStruct((V, N, k2), jnp.float32),
    )(partials, xs, dis, w, b)


def _pad_w1(w_row):
    ch = w_row.shape[-1]
    return jnp.concatenate(
        [jnp.reshape(w_row, (ch, 1)), jnp.zeros((ch, 127), jnp.float32)], axis=1)


def _finish_body(p_ref, xs_ref, dis_ref,
                 cw1_ref, cb1_ref, cw2_ref, cb2_ref,
                 pw1_ref, pb1_ref, pw2_ref, pb2_ref,
                 aw1_ref, ab1_ref, aw2_ref, ab2_ref,
                 fw1_ref, fb1_ref, fw2_ref, fb2_ref,
                 z_ref, pz_ref, logit_ref, fused_ref, flogit_ref, att_ref):
    zs = []
    for v in range(V):
        sacc = p_ref[0, v] + p_ref[1, v] + xs_ref[v]
        z = dis_ref[v] * sacc
        zs.append(z)
        z_ref[v] = z
        hc = jax.nn.relu(jnp.dot(z, cw1_ref[v].T, precision=_HI) + cb1_ref[v])
        logit_ref[v] = jnp.dot(hc, _pad_w1(cw2_ref[v]), precision=_HI)[:, 0:1] + cb2_ref[v, 0]
        hp = jax.nn.relu(jnp.dot(z, pw1_ref[v].T, precision=_HI) + pb1_ref[v])
        pz = jnp.dot(hp, pw2_ref[v].T, precision=_HI) + pb2_ref[v]
        ss = jnp.maximum(jnp.sum(pz * pz, axis=-1, keepdims=True), 1e-24)
        pz_ref[v] = pz * _rsqrt(ss)
    concat = jnp.concatenate(zs, axis=-1)
    ha = jax.nn.relu(jnp.dot(concat, aw1_ref[...].T, precision=_HI) + ab1_ref[...])
    alog = jnp.dot(ha, aw2_ref[...].T, precision=_HI) + ab2_ref[...]
    am = jnp.max(alog, axis=-1, keepdims=True)
    ae = jnp.exp(alog - am)
    att = ae / jnp.sum(ae, axis=-1, keepdims=True)
    att_ref[...] = att
    fused = (zs[0] * att[:, 0:1] + zs[1] * att[:, 1:2] + zs[2] * att[:, 2:3])
    fused_ref[...] = fused
    hf = jax.nn.relu(jnp.dot(fused, fw1_ref[...].T, precision=_HI) + fb1_ref[...])
    flogit_ref[...] = jnp.dot(hf, _pad_w1(fw2_ref[...]), precision=_HI)[:, 0:1] + fb2_ref[0]


def _finish(partials, xs, dis, cw1, cb1, cw2, cb2, pw1, pb1, pw2, pb2,
            aw1, ab1, aw2, ab2, fw1, fb1, fw2, fb2):
    g = N // RB
    k = xs.shape[-1]
    ws = [cw1, cb1, cw2, cb2, pw1, pb1, pw2, pb2, aw1, ab1, aw2, ab2,
          fw1, fb1, fw2, fb2]
    return pl.pallas_call(
        _finish_body,
        grid=(g,),
        in_specs=[
            pl.BlockSpec((NC, V, RB, k), lambda i: (0, 0, i, 0)),
            pl.BlockSpec((V, RB, k), lambda i: (0, i, 0)),
            pl.BlockSpec((V, RB, 1), lambda i: (0, i, 0)),
        ] + [_full(w.shape) for w in ws],
        out_specs=[
            pl.BlockSpec((V, RB, k), lambda i: (0, i, 0)),
            pl.BlockSpec((V, RB, k), lambda i: (0, i, 0)),
            pl.BlockSpec((V, RB, 1), lambda i: (0, i, 0)),
            pl.BlockSpec((RB, k), lambda i: (i, 0)),
            pl.BlockSpec((RB, 1), lambda i: (i, 0)),
            pl.BlockSpec((RB, V), lambda i: (i, 0)),
        ],
        out_shape=[
            jax.ShapeDtypeStruct((V, N, k), jnp.float32),
            jax.ShapeDtypeStruct((V, N, k), jnp.float32),
            jax.ShapeDtypeStruct((V, N, 1), jnp.float32),
            jax.ShapeDtypeStruct((N, k), jnp.float32),
            jax.ShapeDtypeStruct((N, 1), jnp.float32),
            jax.ShapeDtypeStruct((N, V), jnp.float32),
        ],
    )(partials, xs, dis, *ws)


def kernel(node_features, row_ppi, col_ppi, score_ppi, row_path, col_path,
           score_path, row_go, col_go, score_go, enc_W1, enc_b1, enc_W2,
           enc_b2, enc_W3, enc_b3, cls_W1, cls_b1, cls_W2, cls_b2, proj_W1,
           proj_b1, proj_W2, proj_b2, att_W1, att_b1, att_W2, att_b2,
           fus_W1, fus_b1, fus_W2, fus_b2):
    rows = [row_ppi, row_path, row_go]
    cols = [col_ppi, col_path, col_go]
    scores = [score_ppi, score_path, score_go]
    mult1 = NW * EB * 2 * 8
    mult_b = NW * EB * 2 * 8

    ev = [_pad_edges(cols[v] + v * N, rows[v], scores[v], mult1)
          for v in range(V)]
    edges1 = jnp.concatenate([_pack_edges(*e) for e in ev])
    evl = [_pad_edges(cols[v] + v * NP, rows[v], scores[v], mult1)
           for v in range(V)]
    edges_l23 = jnp.concatenate([_pack_edges(*e) for e in evl])
    col_b, row_b, score_b = _pad_edges(
        jnp.concatenate([cols[v] + v * NP for v in range(V)]),
        jnp.concatenate([rows[v] + v * NP for v in range(V)]),
        jnp.concatenate(scores), mult_b)
    edges_b = _pack_edges(col_b, row_b, score_b)
    e1 = ev[0][0].shape[0]
    eb = col_b.shape[0]

    zeros16 = jnp.zeros((V * NP // NS, 16), jnp.float32)
    zeros64 = jnp.zeros((NP // NS, 64), jnp.float32)
    zeros128 = jnp.zeros((NP // NS, 128), jnp.float32)
    ones = jnp.ones((V * NP, 16), jnp.float32)

    def _padr(x):
        return jnp.pad(x, ((0, 0), (0, NP - N), (0, 0))).reshape(V * NP, -1)

    degp = _make_spmm(V * NP, 16, eb)(ones, edges_b, zeros16)
    degp = degp[0].reshape(NC, V, NP, 16)[:, :, :N]

    dis, xs1 = _prep(degp, node_features, enc_W1, enc_b1)

    p1 = _make_spmm(NP, 128, e1, V)(
        xs1.reshape(V * N, 128), edges1, zeros128)
    p1 = jnp.moveaxis(p1[:, :, :N], 0, 1)

    xs2 = _combine(p1, xs1, dis, enc_W2, enc_b2)

    p2 = _make_spmm(NP, 64, e1, V)(_padr(xs2), edges_l23, zeros64)
    xs3 = _combine(jnp.moveaxis(p2[:, :, :N], 0, 1), xs2, dis,
                   enc_W3, enc_b3)

    p3 = _make_spmm(NP, 64, e1, V)(_padr(xs3), edges_l23, zeros64)

    z, pz, logits, fused, flogit, att = _finish(
        jnp.moveaxis(p3[:, :, :N], 0, 1), xs3, dis,
        cls_W1, cls_b1, cls_W2, cls_b2, proj_W1, proj_b1, proj_W2, proj_b2,
        att_W1, att_b1, att_W2, att_b2, fus_W1, fus_b1, fus_W2, fus_b2)

    return (z[0], z[1], z[2], pz[0], pz[1], pz[2],
            logits[0, :, 0], logits[1, :, 0], logits[2, :, 0],
            fused, flogit[:, 0], att)


# speedup vs baseline: 7.3893x; 1.1176x over previous
import functools

import jax
import jax.numpy as jnp
from jax import lax
from jax.experimental import pallas as pl
from jax.experimental.pallas import tpu as pltpu
from jax.experimental.pallas import tpu_sc as plsc

N = 10000
NP = 10112
V = 3
NC = 2
NS = 16
NW = NC * NS
EB = 128

_HI = lax.Precision.DEFAULT
_F0 = 0.65


def _group_size(k):
    return {128: 2, 64: 4}.get(k, 8)


@functools.cache
def _make_spmm(n_rows, k, e_seg, n_seg=1, frac0=None):
    G = _group_size(k)
    per_tile = e_seg // NW
    nbt = per_tile // EB
    if frac0 is None:
        nbt0 = nbt1 = nbt
    else:
        nbt0 = int(round(2 * nbt * frac0 / (2 * G))) * 2 * G
        nbt1 = 2 * nbt - nbt0
    assert nbt0 % (2 * G) == 0 and nbt1 % (2 * G) == 0
    assert nbt0 // G >= 4 and nbt1 // G >= 4
    rpt = n_rows // NS
    mesh = plsc.VectorSubcoreMesh(core_axis_name="c", subcore_axis_name="s")

    @functools.partial(
        pl.kernel,
        out_type=jax.ShapeDtypeStruct((n_seg, NC, n_rows, k), jnp.float32),
        mesh=mesh,
        scratch_types=[
            pltpu.VMEM((3 * G, EB), jnp.int32),
            pltpu.VMEM((3 * G, EB), jnp.int32),
        ] + [pltpu.VMEM((EB, k), jnp.float32) for _ in range(G)] + [
            pltpu.SemaphoreType.DMA((G,)),
            pltpu.SemaphoreType.DMA((G,)),
            pltpu.SemaphoreType.DMA((2,)),
            pltpu.VMEM_SHARED((n_rows, k), jnp.float32),
        ],
        compiler_params=pltpu.CompilerParams(use_tc_tiling_on_sc=False),
    )
    def spmm(x_hbm, edges_hbm, zeros_hbm, out_hbm, eb0, eb1, *rest):
        rows = rest[:G]
        gsem, ssem, esem, acc_sh = rest[G:]
        c = lax.axis_index("c")
        s = lax.axis_index("s")
        nbt_c = jnp.where(c == 0, nbt0, nbt1)
        ng_c = nbt_c // G
        tile_off = jnp.where(c == 0, s * nbt0, NS * nbt0 + s * nbt1)

        def scale(buf, j):
            def body16(g16, c2):
                svec = lax.bitcast_convert_type(
                    buf[3 * j + 2, pl.ds(g16 * 16, 16)], jnp.float32)
                for e in range(16):
                    lane = jnp.full((16,), e, jnp.int32)
                    sv = svec.at[lane].get(mode="promise_in_bounds")
                    r = g16 * 16 + e
                    for jj in range(k // 16):
                        sl = pl.ds(jj * 16, 16)
                        rows[j][r, sl] = rows[j][r, sl] * sv
                return c2
            lax.fori_loop(0, EB // 16, body16, 0)

        def do_group(buf, first):
            descs = []
            for j in range(G):
                if not first:
                    pltpu.make_async_copy(
                        rows[j], acc_sh.at[buf.at[3 * j + 1]],
                        ssem.at[j]).wait()
                descs.append(pltpu.async_copy(
                    x_hbm.at[buf.at[3 * j]], rows[j], gsem.at[j]))
            for j in range(G):
                descs[j].wait()
                scale(buf, j)
                pltpu.async_copy(rows[j], acc_sh.at[buf.at[3 * j + 1]],
                                 ssem.at[j], add=True)

        for seg in range(n_seg):
            pltpu.sync_copy(zeros_hbm, acc_sh.at[pl.ds(s * rpt, rpt)])
            plsc.subcore_barrier()

            b0 = seg * NW * nbt + tile_off

            def edge_issue(grp, buf, p):
                return pltpu.async_copy(
                    edges_hbm.at[pl.ds((b0 + grp * G) * 3, 3 * G)],
                    buf, esem.at[p])

            def edge_wait(buf, p):
                pltpu.make_async_copy(
                    edges_hbm.at[pl.ds(b0 * 3, 3 * G)], buf,
                    esem.at[p]).wait()

            pltpu.sync_copy(edges_hbm.at[pl.ds(b0 * 3, 3 * G)], eb0)
            edge_issue(1, eb1, 1)
            do_group(eb0, first=True)
            edge_issue(2, eb0, 0)

            def pair(i, carry):
                g1 = 1 + 2 * i
                edge_wait(eb1, 1)
                do_group(eb1, first=False)

                @pl.when(g1 + 2 < ng_c)
                def _():
                    edge_issue(g1 + 2, eb1, 1)

                edge_wait(eb0, 0)
                do_group(eb0, first=False)

                @pl.when(g1 + 3 < ng_c)
                def _():
                    edge_issue(g1 + 3, eb0, 0)
                return carry

            lax.fori_loop(0, (ng_c - 2) // 2, pair, 0)

            edge_wait(eb1, 1)
            do_group(eb1, first=False)
            for j in range(G):
                pltpu.make_async_copy(
                    rows[j], acc_sh.at[eb1.at[3 * j + 1]], ssem.at[j]).wait()

            plsc.subcore_barrier()
            pltpu.sync_copy(acc_sh.at[pl.ds(s * rpt, rpt)],
                            out_hbm.at[seg, c, pl.ds(s * rpt, rpt), :])

    return spmm


def _pad_edges(col, row, score, mult):
    e = col.shape[0]
    pad = (-e) % mult
    if pad:
        z = jnp.zeros((pad,), jnp.int32)
        col = jnp.concatenate([col, z])
        row = jnp.concatenate([row, z])
        score = jnp.concatenate([score, jnp.zeros((pad,), score.dtype)])
    return col, row, score


def _pack_edges(col, row, score):
    nb = col.shape[0] // EB
    packed = jnp.stack([
        col.reshape(nb, EB), row.reshape(nb, EB),
        lax.bitcast_convert_type(score, jnp.int32).reshape(nb, EB)], axis=1)
    return packed.reshape(nb * 3, EB)


RB = 1000


def _full(spec_shape):
    r = len(spec_shape)
    return pl.BlockSpec(spec_shape, lambda i: (0,) * r)


def _rsqrt(x):
    r = lax.rsqrt(x)
    r = r * (1.5 - 0.5 * x * r * r)
    r = r * (1.5 - 0.5 * x * r * r)
    return r


def _prep_body(degp_ref, nf_ref, w1_ref, b1_ref, dis_ref, xs1_ref):
    deg = degp_ref[0, :, :, 0] + degp_ref[1, :, :, 0] + 1.0
    dis = _rsqrt(jnp.maximum(deg, 1e-12))
    dis_ref[...] = dis[:, :, None]
    nf = nf_ref[...]
    for v in range(V):
        x1 = jnp.dot(nf, w1_ref[v].T, precision=_HI) + b1_ref[v]
        xs1_ref[v] = dis[v][:, None] * x1


def _prep(degp, nf, w1, b1):
    g = N // RB
    return pl.pallas_call(
        _prep_body,
        grid=(g,),
        in_specs=[
            pl.BlockSpec((NC, V, RB, 16), lambda i: (0, 0, i, 0)),
            pl.BlockSpec((RB, 128), lambda i: (i, 0)),
            _full(w1.shape),
            _full(b1.shape),
        ],
        out_specs=[
            pl.BlockSpec((V, RB, 1), lambda i: (0, i, 0)),
            pl.BlockSpec((V, RB, 128), lambda i: (0, i, 0)),
        ],
        out_shape=[
            jax.ShapeDtypeStruct((V, N, 1), jnp.float32),
            jax.ShapeDtypeStruct((V, N, 128), jnp.float32),
        ],
    )(degp, nf, w1, b1)


def _combine_body(p_ref, xs_ref, dis_ref, w_ref, b_ref, out_ref):
    for v in range(V):
        sacc = p_ref[0, v] + p_ref[1, v] + xs_ref[v]
        h = dis_ref[v] * sacc
        h = jnp.where(h > 0, h, 0.2 * h)
        out_ref[v] = dis_ref[v] * (
            jnp.dot(h, w_ref[v].T, precision=_HI) + b_ref[v])


def _combine(partials, xs, dis, w, b):
    g = N // RB
    k = xs.shape[-1]
    k2 = w.shape[1]
    return pl.pallas_call(
        _combine_body,
        grid=(g,),
        in_specs=[
            pl.BlockSpec((NC, V, RB, k), lambda i: (0, 0, i, 0)),
            pl.BlockSpec((V, RB, k), lambda i: (0, i, 0)),
            pl.BlockSpec((V, RB, 1), lambda i: (0, i, 0)),
            _full(w.shape),
            _full(b.shape),
        ],
        out_specs=pl.BlockSpec((V, RB, k2), lambda i: (0, i, 0)),
        out_shape=jax.ShapeDtypeStruct((V, N, k2), jnp.float32),
    )(partials, xs, dis, w, b)


def _pad_w1(w_row):
    ch = w_row.shape[-1]
    return jnp.concatenate(
        [jnp.reshape(w_row, (ch, 1)), jnp.zeros((ch, 127), jnp.float32)], axis=1)


def _finish_body(p_ref, xs_ref, dis_ref,
                 cw1_ref, cb1_ref, cw2_ref, cb2_ref,
                 pw1_ref, pb1_ref, pw2_ref, pb2_ref,
                 aw1_ref, ab1_ref, aw2_ref, ab2_ref,
                 fw1_ref, fb1_ref, fw2_ref, fb2_ref,
                 z_ref, pz_ref, logit_ref, fused_ref, flogit_ref, att_ref):
    zs = []
    for v in range(V):
        sacc = p_ref[0, v] + p_ref[1, v] + xs_ref[v]
        z = dis_ref[v] * sacc
        zs.append(z)
        z_ref[v] = z
        hc = jax.nn.relu(jnp.dot(z, cw1_ref[v].T, precision=_HI) + cb1_ref[v])
        logit_ref[v] = jnp.dot(hc, _pad_w1(cw2_ref[v]), precision=_HI)[:, 0:1] + cb2_ref[v, 0]
        hp = jax.nn.relu(jnp.dot(z, pw1_ref[v].T, precision=_HI) + pb1_ref[v])
        pz = jnp.dot(hp, pw2_ref[v].T, precision=_HI) + pb2_ref[v]
        ss = jnp.maximum(jnp.sum(pz * pz, axis=-1, keepdims=True), 1e-24)
        pz_ref[v] = pz * _rsqrt(ss)
    concat = jnp.concatenate(zs, axis=-1)
    ha = jax.nn.relu(jnp.dot(concat, aw1_ref[...].T, precision=_HI) + ab1_ref[...])
    alog = jnp.dot(ha, aw2_ref[...].T, precision=_HI) + ab2_ref[...]
    am = jnp.max(alog, axis=-1, keepdims=True)
    ae = jnp.exp(alog - am)
    att = ae / jnp.sum(ae, axis=-1, keepdims=True)
    att_ref[...] = att
    fused = (zs[0] * att[:, 0:1] + zs[1] * att[:, 1:2] + zs[2] * att[:, 2:3])
    fused_ref[...] = fused
    hf = jax.nn.relu(jnp.dot(fused, fw1_ref[...].T, precision=_HI) + fb1_ref[...])
    flogit_ref[...] = jnp.dot(hf, _pad_w1(fw2_ref[...]), precision=_HI)[:, 0:1] + fb2_ref[0]


def _finish(partials, xs, dis, cw1, cb1, cw2, cb2, pw1, pb1, pw2, pb2,
            aw1, ab1, aw2, ab2, fw1, fb1, fw2, fb2):
    g = N // RB
    k = xs.shape[-1]
    ws = [cw1, cb1, cw2, cb2, pw1, pb1, pw2, pb2, aw1, ab1, aw2, ab2,
          fw1, fb1, fw2, fb2]
    return pl.pallas_call(
        _finish_body,
        grid=(g,),
        in_specs=[
            pl.BlockSpec((NC, V, RB, k), lambda i: (0, 0, i, 0)),
            pl.BlockSpec((V, RB, k), lambda i: (0, i, 0)),
            pl.BlockSpec((V, RB, 1), lambda i: (0, i, 0)),
        ] + [_full(w.shape) for w in ws],
        out_specs=[
            pl.BlockSpec((V, RB, k), lambda i: (0, i, 0)),
            pl.BlockSpec((V, RB, k), lambda i: (0, i, 0)),
            pl.BlockSpec((V, RB, 1), lambda i: (0, i, 0)),
            pl.BlockSpec((RB, k), lambda i: (i, 0)),
            pl.BlockSpec((RB, 1), lambda i: (i, 0)),
            pl.BlockSpec((RB, V), lambda i: (i, 0)),
        ],
        out_shape=[
            jax.ShapeDtypeStruct((V, N, k), jnp.float32),
            jax.ShapeDtypeStruct((V, N, k), jnp.float32),
            jax.ShapeDtypeStruct((V, N, 1), jnp.float32),
            jax.ShapeDtypeStruct((N, k), jnp.float32),
            jax.ShapeDtypeStruct((N, 1), jnp.float32),
            jax.ShapeDtypeStruct((N, V), jnp.float32),
        ],
    )(partials, xs, dis, *ws)


def kernel(node_features, row_ppi, col_ppi, score_ppi, row_path, col_path,
           score_path, row_go, col_go, score_go, enc_W1, enc_b1, enc_W2,
           enc_b2, enc_W3, enc_b3, cls_W1, cls_b1, cls_W2, cls_b2, proj_W1,
           proj_b1, proj_W2, proj_b2, att_W1, att_b1, att_W2, att_b2,
           fus_W1, fus_b1, fus_W2, fus_b2):
    rows = [row_ppi, row_path, row_go]
    cols = [col_ppi, col_path, col_go]
    scores = [score_ppi, score_path, score_go]
    mult1 = NW * EB * 2 * 8
    mult_b = NW * EB * 2 * 8

    ev = [_pad_edges(cols[v] + v * N, rows[v], scores[v], mult1)
          for v in range(V)]
    edges1 = jnp.concatenate([_pack_edges(*e) for e in ev])
    evl = [_pad_edges(cols[v] + v * NP, rows[v], scores[v], mult1)
           for v in range(V)]
    edges_l23 = jnp.concatenate([_pack_edges(*e) for e in evl])
    col_b, row_b, score_b = _pad_edges(
        jnp.concatenate([cols[v] + v * NP for v in range(V)]),
        jnp.concatenate([rows[v] + v * NP for v in range(V)]),
        jnp.concatenate(scores), mult_b)
    edges_b = _pack_edges(col_b, row_b, score_b)
    e1 = ev[0][0].shape[0]
    eb = col_b.shape[0]

    zeros16 = jnp.zeros((V * NP // NS, 16), jnp.float32)
    zeros64 = jnp.zeros((NP // NS, 64), jnp.float32)
    zeros128 = jnp.zeros((NP // NS, 128), jnp.float32)
    ones = jnp.ones((V * NP, 16), jnp.float32)

    def _padr(x):
        return jnp.pad(x, ((0, 0), (0, NP - N), (0, 0))).reshape(V * NP, -1)

    degp = _make_spmm(V * NP, 16, eb, frac0=_F0)(ones, edges_b, zeros16)
    degp = degp[0].reshape(NC, V, NP, 16)[:, :, :N]

    dis, xs1 = _prep(degp, node_features, enc_W1, enc_b1)

    p1 = _make_spmm(NP, 128, e1, V, frac0=_F0)(
        xs1.reshape(V * N, 128), edges1, zeros128)
    p1 = jnp.moveaxis(p1[:, :, :N], 0, 1)

    xs2 = _combine(p1, xs1, dis, enc_W2, enc_b2)

    p2 = _make_spmm(NP, 64, e1, V, frac0=_F0)(_padr(xs2), edges_l23, zeros64)
    xs3 = _combine(jnp.moveaxis(p2[:, :, :N], 0, 1), xs2, dis,
                   enc_W3, enc_b3)

    p3 = _make_spmm(NP, 64, e1, V, frac0=_F0)(_padr(xs3), edges_l23, zeros64)

    z, pz, logits, fused, flogit, att = _finish(
        jnp.moveaxis(p3[:, :, :N], 0, 1), xs3, dis,
        cls_W1, cls_b1, cls_W2, cls_b2, proj_W1, proj_b1, proj_W2, proj_b2,
        att_W1, att_b1, att_W2, att_b2, fus_W1, fus_b1, fus_W2, fus_b2)

    return (z[0], z[1], z[2], pz[0], pz[1], pz[2],
            logits[0, :, 0], logits[1, :, 0], logits[2, :, 0],
            fused, flogit[:, 0], att)
